# baseline (speedup 1.0000x reference)
import functools

import jax
import jax.numpy as jnp
from jax import lax
from jax.experimental import pallas as pl
from jax.experimental.pallas import tpu as pltpu
from jax.experimental.pallas import tpu_sc as plsc

_CH = 128
_NCHUNKS = 5


@functools.lru_cache(maxsize=None)
def _make_sc_gather(V: int, D: int, Nc: int, NC: int, NS: int):
    NW = NC * NS
    n_per_w = Nc // NW
    n_ch = n_per_w // _CH

    mesh = plsc.VectorSubcoreMesh(core_axis_name="c", subcore_axis_name="s")

    @functools.partial(
        pl.kernel,
        mesh=mesh,
        out_type=jax.ShapeDtypeStruct((Nc, 2 * D), jnp.float32),
        scratch_types=[
            pltpu.VMEM((n_ch, _CH), jnp.int32),
            pltpu.VMEM((_CH, 2 * D), jnp.float32),
            pltpu.VMEM((_CH, 2 * D), jnp.float32),
            pltpu.SemaphoreType.DMA,
            pltpu.SemaphoreType.DMA,
        ],
    )
    def gather_k(table_hbm, idx_hbm, out_hbm, idx_v, buf0, buf1, sem0, sem1):
        wid = lax.axis_index("s") * NC + lax.axis_index("c")
        base = wid * n_per_w
        pltpu.sync_copy(idx_hbm.at[wid], idx_v)
        bufs = (buf0, buf1)
        sems = (sem0, sem1)

        def issue(j, k):
            pltpu.async_copy(table_hbm.at[idx_v.at[j]], bufs[k], sems[k])

        def finish(j, k):
            pltpu.make_async_copy(
                table_hbm.at[idx_v.at[j]], bufs[k], sems[k]).wait()
            pltpu.sync_copy(bufs[k],
                            out_hbm.at[pl.ds(base + j * _CH, _CH)])

        issue(0, 0)

        def body(p, carry):
            j = 2 * p

            @pl.when(j + 1 < n_ch)
            def _():
                issue(j + 1, 1)

            finish(j, 0)

            @pl.when(j + 2 < n_ch)
            def _():
                issue(j + 2, 0)

            @pl.when(j + 1 < n_ch)
            def _():
                finish(j + 1, 1)

            return carry

        lax.fori_loop(0, (n_ch + 1) // 2, body, 0)

    return gather_k


def _tc_compute(g_ref, vcT_ref, Wc_ref, Wv_ref, bv_ref, bc_ref, gm_ref,
                bt_ref, o_ref):
    _, D, B = o_ref.shape
    f32 = jnp.float32
    u = lax.dot_general(Wv_ref[...], Wc_ref[..., D:],
                        (((0,), (1,)), ((), ())),
                        preferred_element_type=f32)
    eye = jnp.asarray(
        lax.broadcasted_iota(jnp.int32, (D, D), 0)
        == lax.broadcasted_iota(jnp.int32, (D, D), 1), f32)
    ones_db = jnp.ones((D, B), f32)
    dn = (((1,), (0,)), ((), ()))
    colmat = lambda row: lax.dot_general(eye * row, ones_db, dn,
                                         preferred_element_type=f32)
    ball_row = lax.dot_general(bv_ref[...], Wc_ref[..., D:],
                               (((1,), (1,)), ((), ())),
                               preferred_element_type=f32) + bc_ref[...]
    valT = lax.dot_general(u, vcT_ref[...], (((0,), (0,)), ((), ())),
                           preferred_element_type=f32)
    yT = (lax.dot_general(Wc_ref[..., :D], g_ref[..., :D],
                          (((1,), (1,)), ((), ())),
                          preferred_element_type=f32)
          + valT + colmat(ball_row))
    mu = jnp.mean(yT, axis=0, keepdims=True)
    yc = yT - mu
    var = jnp.mean(yc * yc, axis=0, keepdims=True)
    ycr = yc * lax.rsqrt(var + 1e-5)
    G1 = eye * gm_ref[...]
    o_ref[0] = (lax.dot_general(G1, ycr, dn, preferred_element_type=f32)
                + colmat(bt_ref[...]))


def _tc_body_first(g_ref, vcT_ref, Wc_ref, Wv_ref, bv_ref, bc_ref, gm_ref,
                   bt_ref, o_ref):
    _tc_compute(g_ref, vcT_ref, Wc_ref, Wv_ref, bv_ref, bc_ref, gm_ref,
                bt_ref, o_ref)


def _tc_body_chained(g_ref, vcT_ref, Wc_ref, Wv_ref, bv_ref, bc_ref, gm_ref,
                     bt_ref, prev_ref, o_ref):
    del prev_ref
    _tc_compute(g_ref, vcT_ref, Wc_ref, Wv_ref, bv_ref, bc_ref, gm_ref,
                bt_ref, o_ref)


def _dense_chunk(g_c, vcT, Wc, Wv, bv, bc, gamma, beta, B, T, D, c, Tc,
                 prev_out):
    steps = Tc
    grid = (steps,)
    small = lambda shp: pl.BlockSpec(shp, lambda i: tuple(0 for _ in shp))
    in_specs = [
        pl.BlockSpec((B, 2 * D), lambda i: (i, 0)),
        pl.BlockSpec((2, B), lambda i, c=c, s=steps: (0, c * s + i)),
        small(Wc.shape),
        small(Wv.shape),
        small((1, D)),
        small((1, D)),
        small((1, D)),
        small((1, D)),
    ]
    args = [g_c, vcT, Wc, Wv, bv.reshape(1, D), bc.reshape(1, D),
            gamma.reshape(1, D), beta.reshape(1, D)]
    if prev_out is None:
        body = _tc_body_first
        aliases = {}
    else:
        body = _tc_body_chained
        in_specs.append(pl.BlockSpec(memory_space=pl.ANY))
        args.append(prev_out)
        aliases = {8: 0}
    return pl.pallas_call(
        body,
        grid=grid,
        in_specs=in_specs,
        out_specs=pl.BlockSpec((1, D, B),
                               lambda i, c=c, s=steps: (c * s + i, 0, 0)),
        out_shape=jax.ShapeDtypeStruct((T, D, B), jnp.float32),
        input_output_aliases=aliases,
    )(*args)


def kernel(trait_values, trait_confidences, trait_indices, emb_table,
           Wv, bv, Wc, bc, gamma, beta):
    B, T = trait_values.shape
    V, D = emb_table.shape
    N = B * T
    info = plsc.get_sparse_core_info()
    NW = info.num_cores * info.num_subcores
    Tc = T // _NCHUNKS
    Nc = Tc * B
    table_pad = jnp.concatenate(
        [emb_table, jnp.zeros((V, D), jnp.float32)], axis=1)
    idxT = trait_indices.T.astype(jnp.int32)
    gk = _make_sc_gather(V, D, Nc, info.num_cores, info.num_subcores)
    gs = [
        gk(table_pad,
           idxT[c * Tc:(c + 1) * Tc].reshape(NW, Nc // (NW * _CH), _CH))
        for c in range(_NCHUNKS)
    ]
    vcT = jnp.stack(
        [trait_values.T.reshape(N), trait_confidences.T.reshape(N)], axis=0)
    out = None
    for c in range(_NCHUNKS):
        out = _dense_chunk(gs[c], vcT, Wc, Wv, bv, bc, gamma, beta,
                           B, T, D, c, Tc, out)
    return jnp.transpose(out, (2, 0, 1))


# speedup vs baseline: 1.0004x; 1.0004x over previous
import functools

import jax
import jax.numpy as jnp
from jax import lax
from jax.experimental import pallas as pl
from jax.experimental.pallas import tpu as pltpu
from jax.experimental.pallas import tpu_sc as plsc

_CH = 128
_NCHUNKS = 4


@functools.lru_cache(maxsize=None)
def _make_sc_gather(V: int, D: int, Nc: int, NC: int, NS: int):
    NW = NC * NS
    n_per_w = Nc // NW
    n_ch = n_per_w // _CH

    mesh = plsc.VectorSubcoreMesh(core_axis_name="c", subcore_axis_name="s")

    @functools.partial(
        pl.kernel,
        mesh=mesh,
        out_type=jax.ShapeDtypeStruct((Nc, 2 * D), jnp.float32),
        scratch_types=[
            pltpu.VMEM((n_ch, _CH), jnp.int32),
            pltpu.VMEM((_CH, 2 * D), jnp.float32),
            pltpu.VMEM((_CH, 2 * D), jnp.float32),
            pltpu.SemaphoreType.DMA,
            pltpu.SemaphoreType.DMA,
        ],
    )
    def gather_k(table_hbm, idx_hbm, out_hbm, idx_v, buf0, buf1, sem0, sem1):
        wid = lax.axis_index("s") * NC + lax.axis_index("c")
        base = wid * n_per_w
        pltpu.sync_copy(idx_hbm.at[wid], idx_v)
        bufs = (buf0, buf1)
        sems = (sem0, sem1)

        def issue(j, k):
            pltpu.async_copy(table_hbm.at[idx_v.at[j]], bufs[k], sems[k])

        def finish(j, k):
            pltpu.make_async_copy(
                table_hbm.at[idx_v.at[j]], bufs[k], sems[k]).wait()
            pltpu.sync_copy(bufs[k],
                            out_hbm.at[pl.ds(base + j * _CH, _CH)])

        issue(0, 0)

        def body(p, carry):
            j = 2 * p

            @pl.when(j + 1 < n_ch)
            def _():
                issue(j + 1, 1)

            finish(j, 0)

            @pl.when(j + 2 < n_ch)
            def _():
                issue(j + 2, 0)

            @pl.when(j + 1 < n_ch)
            def _():
                finish(j + 1, 1)

            return carry

        lax.fori_loop(0, (n_ch + 1) // 2, body, 0)

    return gather_k


def _tc_compute(g_ref, vcT_ref, Wc_ref, Wv_ref, bv_ref, bc_ref, gm_ref,
                bt_ref, o_ref):
    _, D, B = o_ref.shape
    f32 = jnp.float32
    u = lax.dot_general(Wv_ref[...], Wc_ref[..., D:],
                        (((0,), (1,)), ((), ())),
                        preferred_element_type=f32)
    eye = jnp.asarray(
        lax.broadcasted_iota(jnp.int32, (D, D), 0)
        == lax.broadcasted_iota(jnp.int32, (D, D), 1), f32)
    ones_db = jnp.ones((D, B), f32)
    dn = (((1,), (0,)), ((), ()))
    colmat = lambda row: lax.dot_general(eye * row, ones_db, dn,
                                         preferred_element_type=f32)
    ball_row = lax.dot_general(bv_ref[...], Wc_ref[..., D:],
                               (((1,), (1,)), ((), ())),
                               preferred_element_type=f32) + bc_ref[...]
    valT = lax.dot_general(u, vcT_ref[...], (((0,), (0,)), ((), ())),
                           preferred_element_type=f32)
    yT = (lax.dot_general(Wc_ref[..., :D], g_ref[..., :D],
                          (((1,), (1,)), ((), ())),
                          preferred_element_type=f32)
          + valT + colmat(ball_row))
    mu = jnp.mean(yT, axis=0, keepdims=True)
    yc = yT - mu
    var = jnp.mean(yc * yc, axis=0, keepdims=True)
    ycr = yc * lax.rsqrt(var + 1e-5)
    G1 = eye * gm_ref[...]
    o_ref[0] = (lax.dot_general(G1, ycr, dn, preferred_element_type=f32)
                + colmat(bt_ref[...]))


def _tc_body_first(g_ref, vcT_ref, Wc_ref, Wv_ref, bv_ref, bc_ref, gm_ref,
                   bt_ref, o_ref):
    _tc_compute(g_ref, vcT_ref, Wc_ref, Wv_ref, bv_ref, bc_ref, gm_ref,
                bt_ref, o_ref)


def _tc_body_chained(g_ref, vcT_ref, Wc_ref, Wv_ref, bv_ref, bc_ref, gm_ref,
                     bt_ref, prev_ref, o_ref):
    del prev_ref
    _tc_compute(g_ref, vcT_ref, Wc_ref, Wv_ref, bv_ref, bc_ref, gm_ref,
                bt_ref, o_ref)


def _dense_chunk(g_c, vcT, Wc, Wv, bv, bc, gamma, beta, B, T, D, c, Tc,
                 prev_out):
    steps = Tc
    grid = (steps,)
    small = lambda shp: pl.BlockSpec(shp, lambda i: tuple(0 for _ in shp))
    in_specs = [
        pl.BlockSpec((B, 2 * D), lambda i: (i, 0)),
        pl.BlockSpec((2, B), lambda i, c=c, s=steps: (0, c * s + i)),
        small(Wc.shape),
        small(Wv.shape),
        small((1, D)),
        small((1, D)),
        small((1, D)),
        small((1, D)),
    ]
    args = [g_c, vcT, Wc, Wv, bv.reshape(1, D), bc.reshape(1, D),
            gamma.reshape(1, D), beta.reshape(1, D)]
    if prev_out is None:
        body = _tc_body_first
        aliases = {}
    else:
        body = _tc_body_chained
        in_specs.append(pl.BlockSpec(memory_space=pl.ANY))
        args.append(prev_out)
        aliases = {8: 0}
    return pl.pallas_call(
        body,
        grid=grid,
        in_specs=in_specs,
        out_specs=pl.BlockSpec((1, D, B),
                               lambda i, c=c, s=steps: (c * s + i, 0, 0)),
        out_shape=jax.ShapeDtypeStruct((T, D, B), jnp.float32),
        input_output_aliases=aliases,
    )(*args)


def kernel(trait_values, trait_confidences, trait_indices, emb_table,
           Wv, bv, Wc, bc, gamma, beta):
    B, T = trait_values.shape
    V, D = emb_table.shape
    N = B * T
    info = plsc.get_sparse_core_info()
    NW = info.num_cores * info.num_subcores
    Tc = T // _NCHUNKS
    Nc = Tc * B
    table_pad = jnp.concatenate(
        [emb_table, jnp.zeros((V, D), jnp.float32)], axis=1)
    idxT = trait_indices.T.astype(jnp.int32)
    gk = _make_sc_gather(V, D, Nc, info.num_cores, info.num_subcores)
    gs = [
        gk(table_pad,
           idxT[c * Tc:(c + 1) * Tc].reshape(NW, Nc // (NW * _CH), _CH))
        for c in range(_NCHUNKS)
    ]
    vcT = jnp.stack(
        [trait_values.T.reshape(N), trait_confidences.T.reshape(N)], axis=0)
    out = None
    for c in range(_NCHUNKS):
        out = _dense_chunk(gs[c], vcT, Wc, Wv, bv, bc, gamma, beta,
                           B, T, D, c, Tc, out)
    return jnp.transpose(out, (2, 0, 1))
